# baseline (speedup 1.0000x reference)
import functools

import jax
import jax.numpy as jnp
from jax import lax
from jax.experimental import pallas as pl
from jax.experimental.pallas import tpu as pltpu
from jax.experimental.pallas import tpu_sc as plsc

VOCAB = 1000000
DIM = 64
BATCH = 16384
HIST = 50

_INFO = plsc.get_sparse_core_info()
NC = _INFO.num_cores
NS = _INFO.num_subcores
NW = NC * NS
L = 16

BT = BATCH // 128
TPW = BT // NW
TPR = 2
ROWS = TPR * 128
ROUNDS = HIST * (TPW // TPR)
DT = DIM // 8
OUT_ROWS = HIST * DT * BT * 8

_mesh = plsc.VectorSubcoreMesh(core_axis_name="c", subcore_axis_name="s")


@functools.partial(
    pl.kernel,
    out_type=jax.ShapeDtypeStruct((OUT_ROWS, 128), jnp.float32),
    mesh=_mesh,
    scratch_types=[
        pltpu.VMEM((TPR, 128), jnp.int32),
        pltpu.VMEM((TPR, 128), jnp.int32),
        pltpu.VMEM((ROWS, DIM), jnp.float32),
        pltpu.VMEM((ROWS, DIM), jnp.float32),
        pltpu.VMEM((128, 128), jnp.float32),
        pltpu.VMEM((128, 128), jnp.float32),
        pltpu.SemaphoreType.DMA,
        pltpu.SemaphoreType.DMA,
        pltpu.SemaphoreType.DMA,
        pltpu.SemaphoreType.DMA,
    ],
    compiler_params=pltpu.CompilerParams(
        use_tc_tiling_on_sc=False, needs_layout_passes=False),
)
def _gather_kernel(xt_hbm, table_hbm, out_hbm,
                   ix0, ix1, gb0, gb1, st0, st1, g0, g1, s0, s1):
    wid = lax.axis_index("s") * NC + lax.axis_index("c")
    ixs = (ix0, ix1)
    gbs = (gb0, gb1)
    sts = (st0, st1)
    gsems = (g0, g1)
    ssems = (s0, s1)

    iota = lax.iota(jnp.int32, L)
    rvecs = [[iota + (tjloc * 128 + c * L) for c in range(8)]
             for tjloc in range(TPR)]

    def idx_row(r):
        h = r // TPR
        p = r % TPR
        return h * 128 + TPW * wid + TPR * p

    def fire_gathers(r, b):
        for jj in range(TPR):
            pltpu.async_copy(
                table_hbm.at[ixs[b].at[jj]],
                gbs[b].at[pl.ds(jj * 128, 128)],
                gsems[b],
            )

    def drain_gathers(r, b):
        for jj in range(TPR):
            pltpu.make_async_copy(
                table_hbm.at[ixs[b].at[jj]],
                gbs[b].at[pl.ds(jj * 128, 128)],
                gsems[b],
            ).wait()

    def out_base(r, ti):
        h = r // TPR
        p = r % TPR
        return ((h * DT + ti) * BT + TPW * wid + TPR * p) * 8

    def fire_store(r, ti, b):
        pltpu.async_copy(
            sts[b].at[pl.ds(ti * 2 * 8, TPR * 8)],
            out_hbm.at[pl.ds(out_base(r, ti), TPR * 8)],
            ssems[b],
        )

    def wait_stores(r, b):
        for ti in range(DT):
            pltpu.make_async_copy(
                sts[b].at[pl.ds(ti * 2 * 8, TPR * 8)],
                out_hbm.at[pl.ds(out_base(r, ti), TPR * 8)],
                ssems[b],
            ).wait()

    for b in range(2):
        pltpu.sync_copy(xt_hbm.at[pl.ds(idx_row(b), TPR)], ixs[b])
        fire_gathers(b, b)

    @pl.loop(0, ROUNDS, step=2)
    def _rpair(rr):
      for b in range(2):
        r = rr + b
        drain_gathers(r, b)

        @pl.when(r >= 2)
        def _():
            wait_stores(r - 2, b)

        @pl.loop(0, DT)
        def _ti(ti):
            for tjloc in range(TPR):
                for sub in range(8):
                    srow = ti * (TPR * 8) + tjloc * 8 + sub
                    d = ti * 8 + sub
                    for c in range(8):
                        v = plsc.load_gather(
                            gbs[b], [rvecs[tjloc][c], jnp.full((L,), d, jnp.int32)])
                        sts[b][srow, pl.ds(c * L, L)] = v

        for ti in range(DT):
            fire_store(r, ti, b)

        @pl.when(r + 2 < ROUNDS)
        def _():
            pltpu.sync_copy(xt_hbm.at[pl.ds(idx_row(r + 2), TPR)], ixs[b])
            fire_gathers(r + 2, b)

    for r in (ROUNDS - 2, ROUNDS - 1):
        wait_stores(r, r % 2)


def kernel(x, table):
    xt = x.T.reshape(HIST * BT, 128).astype(jnp.int32)
    out2 = _gather_kernel(xt, table)
    out5 = out2.reshape(HIST, DT, BT, 8, 128)
    return out5.transpose(2, 4, 0, 1, 3).reshape(BATCH, HIST, DIM)


# speedup vs baseline: 1.1730x; 1.1730x over previous
import functools

import jax
import jax.numpy as jnp
from jax import lax
from jax.experimental import pallas as pl
from jax.experimental.pallas import tpu as pltpu
from jax.experimental.pallas import tpu_sc as plsc

VOCAB = 1000000
DIM = 64
BATCH = 16384
HIST = 50

_INFO = plsc.get_sparse_core_info()
NC = _INFO.num_cores
NS = _INFO.num_subcores
NW = NC * NS
L = 16

BT = BATCH // 128
TPW = BT // NW
TPR = 2
ROWS = TPR * 128
ROUNDS = HIST * (TPW // TPR)
DT = DIM // 8
OUT_ROWS = HIST * DT * BT * 8

_mesh = plsc.VectorSubcoreMesh(core_axis_name="c", subcore_axis_name="s")


@functools.partial(
    pl.kernel,
    out_type=jax.ShapeDtypeStruct((OUT_ROWS, 128), jnp.float32),
    mesh=_mesh,
    scratch_types=[
        pltpu.VMEM((TPR, 128), jnp.int32),
        pltpu.VMEM((TPR, 128), jnp.int32),
        pltpu.VMEM((ROWS, DIM), jnp.float32),
        pltpu.VMEM((ROWS, DIM), jnp.float32),
        pltpu.VMEM((128, 128), jnp.float32),
        pltpu.VMEM((128, 128), jnp.float32),
        pltpu.SemaphoreType.DMA,
        pltpu.SemaphoreType.DMA,
        pltpu.SemaphoreType.DMA,
        pltpu.SemaphoreType.DMA,
    ],
    compiler_params=pltpu.CompilerParams(
        use_tc_tiling_on_sc=False, needs_layout_passes=False),
)
def _gather_kernel(xt_hbm, table_hbm, out_hbm,
                   ix0, ix1, gb0, gb1, st0, st1, g0, g1, s0, s1):
    wid = lax.axis_index("s") * NC + lax.axis_index("c")
    ixs = (ix0, ix1)
    gbs = (gb0, gb1)
    sts = (st0, st1)
    gsems = (g0, g1)
    ssems = (s0, s1)

    iota = lax.iota(jnp.int32, L)
    rvecs = [[iota + (tjloc * 128 + c * L) for c in range(8)]
             for tjloc in range(TPR)]

    def idx_row(r):
        h = r // TPR
        p = r % TPR
        return h * 128 + TPW * wid + TPR * p

    def fire_gathers(r, b):
        for jj in range(TPR):
            pltpu.async_copy(
                table_hbm.at[ixs[b].at[jj]],
                gbs[b].at[pl.ds(jj * 128, 128)],
                gsems[b],
            )

    def drain_gathers(r, b):
        for jj in range(TPR):
            pltpu.make_async_copy(
                table_hbm.at[ixs[b].at[jj]],
                gbs[b].at[pl.ds(jj * 128, 128)],
                gsems[b],
            ).wait()

    def out_base(r, ti):
        h = r // TPR
        p = r % TPR
        return ((h * DT + ti) * BT + TPW * wid + TPR * p) * 8

    def fire_store(r, ti, b):
        pltpu.async_copy(
            sts[b].at[pl.ds(ti * 2 * 8, TPR * 8)],
            out_hbm.at[pl.ds(out_base(r, ti), TPR * 8)],
            ssems[b],
        )

    def wait_stores(r, b):
        for ti in range(DT):
            pltpu.make_async_copy(
                sts[b].at[pl.ds(ti * 2 * 8, TPR * 8)],
                out_hbm.at[pl.ds(out_base(r, ti), TPR * 8)],
                ssems[b],
            ).wait()

    for b in range(2):
        pltpu.sync_copy(xt_hbm.at[pl.ds(idx_row(b), TPR)], ixs[b])
        fire_gathers(b, b)

    @pl.loop(0, ROUNDS, step=2)
    def _rpair(rr):
      for b in range(2):
        r = rr + b
        drain_gathers(r, b)

        @pl.when(r >= 2)
        def _():
            wait_stores(r - 2, b)

        for ti in range(DT):
            for tjloc in range(TPR):
                for sub in range(8):
                    srow = ti * (TPR * 8) + tjloc * 8 + sub
                    d = ti * 8 + sub
                    dvec = jnp.full((L,), d, jnp.int32)
                    vs = [plsc.load_gather(gbs[b], [rvecs[tjloc][c], dvec])
                          for c in range(8)]
                    for c in range(8):
                        sts[b][srow, pl.ds(c * L, L)] = vs[c]
            fire_store(r, ti, b)

        @pl.when(r + 2 < ROUNDS)
        def _():
            pltpu.sync_copy(xt_hbm.at[pl.ds(idx_row(r + 2), TPR)], ixs[b])
            fire_gathers(r + 2, b)

    for r in (ROUNDS - 2, ROUNDS - 1):
        wait_stores(r, r % 2)


def kernel(x, table):
    xt = x.T.reshape(HIST * BT, 128).astype(jnp.int32)
    out2 = _gather_kernel(xt, table)
    out5 = out2.reshape(HIST, DT, BT, 8, 128)
    return out5.transpose(2, 4, 0, 1, 3).reshape(BATCH, HIST, DIM)


# speedup vs baseline: 1.8362x; 1.5654x over previous
import functools

import jax
import jax.numpy as jnp
from jax import lax
from jax.experimental import pallas as pl
from jax.experimental.pallas import tpu as pltpu
from jax.experimental.pallas import tpu_sc as plsc

VOCAB = 1000000
DIM = 64
BATCH = 16384
HIST = 50

_INFO = plsc.get_sparse_core_info()
NC = _INFO.num_cores
NS = _INFO.num_subcores
NW = NC * NS
L = 16

BT = BATCH // 128
TPW = BT // NW
TPR = 2
ROWS = TPR * 128
ROUNDS = HIST * (TPW // TPR)
DT = DIM // 8
OUT_ROWS = HIST * DT * BT * 8

_mesh = plsc.VectorSubcoreMesh(core_axis_name="c", subcore_axis_name="s")


@functools.partial(
    pl.kernel,
    out_type=jax.ShapeDtypeStruct((OUT_ROWS, 128), jnp.float32),
    mesh=_mesh,
    scratch_types=[
        pltpu.VMEM((TPR, 128), jnp.int32),
        pltpu.VMEM((TPR, 128), jnp.int32),
        pltpu.VMEM((ROWS, DIM), jnp.float32),
        pltpu.VMEM((ROWS, DIM), jnp.float32),
        pltpu.VMEM((ROWS, DIM + 1), jnp.float32),
        pltpu.VMEM((128, 128), jnp.float32),
        pltpu.VMEM((128, 128), jnp.float32),
        pltpu.SemaphoreType.DMA,
        pltpu.SemaphoreType.DMA,
        pltpu.SemaphoreType.DMA,
        pltpu.SemaphoreType.DMA,
    ],
    compiler_params=pltpu.CompilerParams(
        use_tc_tiling_on_sc=False, needs_layout_passes=False),
)
def _gather_kernel(xt_hbm, table_hbm, out_hbm,
                   ix0, ix1, gb0, gb1, sg, st0, st1, g0, g1, s0, s1):
    wid = lax.axis_index("s") * NC + lax.axis_index("c")
    ixs = (ix0, ix1)
    gbs = (gb0, gb1)
    sts = (st0, st1)
    gsems = (g0, g1)
    ssems = (s0, s1)

    iota = lax.iota(jnp.int32, L)
    rvecs = [[iota + (tjloc * 128 + c * L) for c in range(8)]
             for tjloc in range(TPR)]

    def idx_row(r):
        h = r // TPR
        p = r % TPR
        return h * 128 + TPW * wid + TPR * p

    def fire_gathers(r, b):
        for jj in range(TPR):
            pltpu.async_copy(
                table_hbm.at[ixs[b].at[jj]],
                gbs[b].at[pl.ds(jj * 128, 128)],
                gsems[b],
            )

    def drain_gathers(r, b):
        for jj in range(TPR):
            pltpu.make_async_copy(
                table_hbm.at[ixs[b].at[jj]],
                gbs[b].at[pl.ds(jj * 128, 128)],
                gsems[b],
            ).wait()

    def out_base(r, ti):
        h = r // TPR
        p = r % TPR
        return ((h * DT + ti) * BT + TPW * wid + TPR * p) * 8

    def fire_store(r, ti, b):
        pltpu.async_copy(
            sts[b].at[pl.ds(ti * 2 * 8, TPR * 8)],
            out_hbm.at[pl.ds(out_base(r, ti), TPR * 8)],
            ssems[b],
        )

    def wait_stores(r, b):
        for ti in range(DT):
            pltpu.make_async_copy(
                sts[b].at[pl.ds(ti * 2 * 8, TPR * 8)],
                out_hbm.at[pl.ds(out_base(r, ti), TPR * 8)],
                ssems[b],
            ).wait()

    for b in range(2):
        pltpu.sync_copy(xt_hbm.at[pl.ds(idx_row(b), TPR)], ixs[b])
        fire_gathers(b, b)

    @pl.loop(0, ROUNDS, step=2)
    def _rpair(rr):
      for b in range(2):
        r = rr + b
        drain_gathers(r, b)

        @pl.when(r >= 2)
        def _():
            wait_stores(r - 2, b)

        @pl.loop(0, ROWS // 8)
        def _cp(row8):
            vs = []
            for k in range(8):
                for c in range(DIM // L):
                    vs.append((k, c, gbs[b][row8 * 8 + k, pl.ds(c * L, L)]))
            for k, c, v in vs:
                sg[row8 * 8 + k, pl.ds(c * L, L)] = v

        @pl.loop(0, DT)
        def _ti(ti):
            for tjloc in range(TPR):
                for sub in range(8):
                    srow = ti * (TPR * 8) + tjloc * 8 + sub
                    dvec = jnp.full((L,), ti * 8 + sub, jnp.int32)
                    vs = [plsc.load_gather(sg, [rvecs[tjloc][c], dvec])
                          for c in range(8)]
                    for c in range(8):
                        sts[b][srow, pl.ds(c * L, L)] = vs[c]
            fire_store(r, ti, b)

        @pl.when(r + 2 < ROUNDS)
        def _():
            pltpu.sync_copy(xt_hbm.at[pl.ds(idx_row(r + 2), TPR)], ixs[b])
            fire_gathers(r + 2, b)

    for r in (ROUNDS - 2, ROUNDS - 1):
        wait_stores(r, r % 2)


def kernel(x, table):
    xt = x.T.reshape(HIST * BT, 128).astype(jnp.int32)
    out2 = _gather_kernel(xt, table)
    out5 = out2.reshape(HIST, DT, BT, 8, 128)
    return out5.transpose(2, 4, 0, 1, 3).reshape(BATCH, HIST, DIM)
